# baseline (speedup 1.0000x reference)
import jax
import jax.numpy as jnp
from jax.experimental import pallas as pl
from jax.experimental.pallas import tpu as pltpu

_BM = 512


def _fused_kernel(z_ref, l_ref, w1_ref, w2_ref, w02_ref, b_ref, out_ref,
                  x0_ref, y_ref):
    p = pl.program_id(0)
    j = pl.program_id(1)
    V = z_ref.shape[1]

    @pl.when((p == 0) & (j == 0))
    def _transpose_in():
        x0_ref[pl.ds(0, V), :] = z_ref[...].T

    x0_blk = x0_ref[pl.ds(j * _BM, _BM), :]

    @pl.when(p == 0)
    def _phase0():
        x1 = jnp.dot(l_ref[...], x0_ref[pl.ds(0, V), :],
                     preferred_element_type=jnp.float32)
        y_ref[pl.ds(j * _BM, _BM), :] = (
            jnp.dot(x0_blk, w1_ref[...], preferred_element_type=jnp.float32)
            + 2.0 * jnp.dot(x1, w2_ref[...], preferred_element_type=jnp.float32)
        )

    @pl.when(p == 1)
    def _phase1():
        r = (
            jnp.dot(l_ref[...], y_ref[pl.ds(0, V), :],
                    preferred_element_type=jnp.float32)
            + jnp.dot(x0_blk, w02_ref[...], preferred_element_type=jnp.float32)
            + b_ref[...]
        )
        out_ref[...] = r.T


def kernel(x, laplacian, weight, bias):
    B, Cin, V = x.shape
    K, _, Cout = weight.shape
    N = B * Cin

    z = x.reshape(N, V)
    w0, w1, w2 = weight[0], weight[1], weight[2]
    w02 = w0 - w2
    b2 = bias.reshape(1, Cout)

    g = -(-V // _BM)
    vpad = g * _BM
    grid = (2, g)
    out = pl.pallas_call(
        _fused_kernel,
        grid=grid,
        in_specs=[
            pl.BlockSpec((N, V), lambda p, j: (0, 0)),
            pl.BlockSpec((_BM, V), lambda p, j: (j, 0)),
            pl.BlockSpec((Cin, Cout), lambda p, j: (0, 0)),
            pl.BlockSpec((Cin, Cout), lambda p, j: (0, 0)),
            pl.BlockSpec((Cin, Cout), lambda p, j: (0, 0)),
            pl.BlockSpec((1, Cout), lambda p, j: (0, 0)),
        ],
        out_specs=pl.BlockSpec((Cout, _BM), lambda p, j: (0, p * j)),
        out_shape=jax.ShapeDtypeStruct((Cout, V), jnp.float32),
        scratch_shapes=[
            pltpu.VMEM((vpad, N), jnp.float32),
            pltpu.VMEM((vpad, Cout), jnp.float32),
        ],
    )(z, laplacian, w1, w2, w02, b2)

    return out.reshape(B, Cout, V)


# speedup vs baseline: 1.0365x; 1.0365x over previous
import jax
import jax.numpy as jnp
from jax.experimental import pallas as pl
from jax.experimental.pallas import tpu as pltpu

_BM = 400


def _fused_kernel(z_ref, l_ref, w1_ref, w2_ref, w02_ref, b_ref, out_ref,
                  x0_ref, y_ref):
    p = pl.program_id(0)
    j = pl.program_id(1)

    @pl.when((p == 0) & (j == 0))
    def _transpose_in():
        x0_ref[...] = z_ref[...].T

    x0_blk = x0_ref[pl.ds(j * _BM, _BM), :]

    @pl.when(p == 0)
    def _phase0():
        x1 = jnp.dot(l_ref[...], x0_ref[...], preferred_element_type=jnp.float32)
        y_ref[pl.ds(j * _BM, _BM), :] = (
            jnp.dot(x0_blk, w1_ref[...], preferred_element_type=jnp.float32)
            + 2.0 * jnp.dot(x1, w2_ref[...], preferred_element_type=jnp.float32)
        )

    @pl.when(p == 1)
    def _phase1():
        out_ref[...] = (
            jnp.dot(l_ref[...], y_ref[...], preferred_element_type=jnp.float32)
            + jnp.dot(x0_blk, w02_ref[...], preferred_element_type=jnp.float32)
            + b_ref[...]
        )


def kernel(x, laplacian, weight, bias):
    B, Cin, V = x.shape
    K, _, Cout = weight.shape
    N = B * Cin

    z = x.reshape(N, V)
    w0, w1, w2 = weight[0], weight[1], weight[2]
    w02 = w0 - w2
    b2 = bias.reshape(1, Cout)

    grid = (2, V // _BM)
    out = pl.pallas_call(
        _fused_kernel,
        grid=grid,
        in_specs=[
            pl.BlockSpec((N, V), lambda p, j: (0, 0)),
            pl.BlockSpec((_BM, V), lambda p, j: (j, 0)),
            pl.BlockSpec((Cin, Cout), lambda p, j: (0, 0)),
            pl.BlockSpec((Cin, Cout), lambda p, j: (0, 0)),
            pl.BlockSpec((Cin, Cout), lambda p, j: (0, 0)),
            pl.BlockSpec((1, Cout), lambda p, j: (0, 0)),
        ],
        out_specs=pl.BlockSpec((_BM, Cout), lambda p, j: (p * j, 0)),
        out_shape=jax.ShapeDtypeStruct((V, Cout), jnp.float32),
        scratch_shapes=[
            pltpu.VMEM((V, N), jnp.float32),
            pltpu.VMEM((V, Cout), jnp.float32),
        ],
    )(z, laplacian, w1, w2, w02, b2)

    return out.T.reshape(B, Cout, V)


# speedup vs baseline: 1.0671x; 1.0295x over previous
import jax
import jax.numpy as jnp
from jax.experimental import pallas as pl
from jax.experimental.pallas import tpu as pltpu

_BM = 400


def _fused_kernel(x0_ref, l_ref, w1_ref, w2_ref, w02_ref, b_ref, out_ref,
                  y_ref):
    p = pl.program_id(0)
    j = pl.program_id(1)
    x0_blk = x0_ref[pl.ds(j * _BM, _BM), :]

    @pl.when(p == 0)
    def _phase0():
        x1 = jnp.dot(l_ref[...], x0_ref[...], preferred_element_type=jnp.float32)
        y_ref[pl.ds(j * _BM, _BM), :] = (
            jnp.dot(x0_blk, w1_ref[...], preferred_element_type=jnp.float32)
            + 2.0 * jnp.dot(x1, w2_ref[...], preferred_element_type=jnp.float32)
        )

    @pl.when(p == 1)
    def _phase1():
        out_ref[...] = (
            jnp.dot(l_ref[...], y_ref[...], preferred_element_type=jnp.float32)
            + jnp.dot(x0_blk, w02_ref[...], preferred_element_type=jnp.float32)
            + b_ref[...]
        )


def kernel(x, laplacian, weight, bias):
    B, Cin, V = x.shape
    K, _, Cout = weight.shape
    N = B * Cin

    x0 = x.reshape(N, V).T
    w0, w1, w2 = weight[0], weight[1], weight[2]
    w02 = w0 - w2
    b2 = bias.reshape(1, Cout)

    grid = (2, V // _BM)
    out = pl.pallas_call(
        _fused_kernel,
        grid=grid,
        in_specs=[
            pl.BlockSpec((V, N), lambda p, j: (0, 0)),
            pl.BlockSpec((_BM, V), lambda p, j: (j, 0)),
            pl.BlockSpec((Cin, Cout), lambda p, j: (0, 0)),
            pl.BlockSpec((Cin, Cout), lambda p, j: (0, 0)),
            pl.BlockSpec((Cin, Cout), lambda p, j: (0, 0)),
            pl.BlockSpec((1, Cout), lambda p, j: (0, 0)),
        ],
        out_specs=pl.BlockSpec((_BM, Cout), lambda p, j: (p * j, 0)),
        out_shape=jax.ShapeDtypeStruct((V, Cout), jnp.float32),
        scratch_shapes=[
            pltpu.VMEM((V, Cout), jnp.float32),
        ],
    )(x0, laplacian, w1, w2, w02, b2)

    return out.T.reshape(B, Cout, V)
